# baseline (speedup 1.0000x reference)
import functools

import jax
import jax.numpy as jnp
from jax import lax
from jax.experimental import pallas as pl
from jax.experimental.pallas import tpu as pltpu
from jax.experimental.pallas import tpu_sc as plsc

B = 16384
D = 32
PACK = 128 // D * D // D
NC = 2
NS = 16
NW = NC * NS
BPW = B // NW
CH = 128
NCHUNK = BPW // CH
LANES = 16


def _gmf_body(ut4, it4, ui, ii, out, uidx_v, iidx_v, urow_v, irow_v,
              gu0, gu1, gi0, gi1, outbuf, semu0, semu1, semi0, semi1):
    wid = lax.axis_index("s") * NC + lax.axis_index("c")
    base = wid * BPW
    pltpu.sync_copy(ui.at[pl.ds(base, BPW)], uidx_v)
    pltpu.sync_copy(ii.at[pl.ds(base, BPW)], iidx_v)

    for k in range(BPW // LANES):
        sl = pl.ds(k * LANES, LANES)
        r = k * LANES // CH
        csl = pl.ds(k * LANES % CH, LANES)
        urow_v[r, csl] = jax.lax.shift_right_logical(uidx_v[sl], 2)
        irow_v[r, csl] = jax.lax.shift_right_logical(iidx_v[sl], 2)

    gu = (gu0, gu1)
    gi = (gi0, gi1)
    semu = (semu0, semu1)
    semi = (semi0, semi1)

    def start(c):
        b = c % 2
        cu = pltpu.async_copy(ut4.at[urow_v.at[c]], gu[b], semu[b])
        ci = pltpu.async_copy(it4.at[irow_v.at[c]], gi[b], semi[b])
        return cu, ci

    def compute(c):
        b = c % 2
        lane = lax.iota(jnp.int32, 16)

        def body(g, carry):
            rows = g * LANES + lane
            sl = pl.ds(c * CH + g * LANES, LANES)
            uoff = (uidx_v[sl] & 3) * D
            ioff = (iidx_v[sl] & 3) * D
            orow = c * CH + rows
            brow = jax.lax.shift_right_logical(orow, 2)
            bcol = (orow & 3) * D
            for j in range(D):
                uv = plsc.load_gather(gu[b], [rows, uoff + j])
                iv = plsc.load_gather(gi[b], [rows, ioff + j])
                plsc.store_scatter(outbuf, [brow, bcol + j], uv * iv)
            return carry

        lax.fori_loop(0, CH // LANES, body, 0)

    pend = start(0)
    for c in range(NCHUNK):
        nxt = start(c + 1) if c + 1 < NCHUNK else None
        pend[0].wait()
        pend[1].wait()
        compute(c)
        pend = nxt

    pltpu.sync_copy(outbuf, out.at[pl.ds(wid * (BPW * D // 128), BPW * D // 128)])


def kernel(user_table, item_table, user_indices, item_indices):
    ut4 = user_table.reshape(-1, 128)
    it4 = item_table.reshape(-1, 128)
    mesh = plsc.VectorSubcoreMesh(core_axis_name="c", subcore_axis_name="s")
    k = functools.partial(
        pl.kernel,
        mesh=mesh,
        out_type=jax.ShapeDtypeStruct((B * D // 128, 128), jnp.float32),
        compiler_params=pltpu.CompilerParams(needs_layout_passes=False),
        scratch_types=[
            pltpu.VMEM((BPW,), jnp.int32),
            pltpu.VMEM((BPW,), jnp.int32),
            pltpu.VMEM((NCHUNK, CH), jnp.int32),
            pltpu.VMEM((NCHUNK, CH), jnp.int32),
            pltpu.VMEM((CH, 128), jnp.float32),
            pltpu.VMEM((CH, 128), jnp.float32),
            pltpu.VMEM((CH, 128), jnp.float32),
            pltpu.VMEM((CH, 128), jnp.float32),
            pltpu.VMEM((BPW * D // 128, 128), jnp.float32),
            pltpu.SemaphoreType.DMA,
            pltpu.SemaphoreType.DMA,
            pltpu.SemaphoreType.DMA,
            pltpu.SemaphoreType.DMA,
        ],
    )(_gmf_body)
    return k(ut4, it4, user_indices, item_indices).reshape(B, D)
